# Initial kernel scaffold; baseline (speedup 1.0000x reference)
import jax
import jax.numpy as jnp
from jax.experimental import pallas as pl


def kernel(x, edge_index, batch, W_in, b_in, W_c1, b_c1, W_c2, b_c2, W_out, b_out):
    raise NotImplementedError("write your pallas kernel here")


# speedup vs baseline: 8.4819x; 8.4819x over previous
import functools

import jax
import jax.numpy as jnp
from jax import lax
from jax.experimental import pallas as pl
from jax.experimental.pallas import tpu as pltpu
from jax.experimental.pallas import tpu_sc as plsc

_N = 10000
_E = 320000
_D = 128
_G = 64
_NC = 2
_NS = 16
_NW = _NC * _NS
_NPAD = 10240
_RPT = _NPAD // _NS
_NCH = 80
_EPW = _NCH * 128
_EPAD = _EPW * _NW
_BLK = 512
_NBLK = _NPAD // _BLK

_mesh = plsc.VectorSubcoreMesh(core_axis_name="c", subcore_axis_name="s")


@functools.partial(
    pl.kernel,
    out_type=jax.ShapeDtypeStruct((_NC, _NPAD), jnp.float32),
    mesh=_mesh,
    scratch_types=[
        pltpu.VMEM((_NCH, 128), jnp.int32),
        pltpu.VMEM((_RPT,), jnp.float32),
        pltpu.VMEM((128,), jnp.float32),
        pltpu.VMEM_SHARED((_NPAD,), jnp.float32),
    ],
)
def _sc_degree(dst_hbm, out_hbm, dstv, zbuf, ones, deg_sh):
    c = lax.axis_index("c")
    s = lax.axis_index("s")
    wid = c * _NS + s

    def _z(i, _):
        zbuf[pl.ds(i * 16, 16)] = jnp.zeros((16,), jnp.float32)
        return 0

    lax.fori_loop(0, _RPT // 16, _z, 0)

    def _o(i, _):
        ones[pl.ds(i * 16, 16)] = jnp.ones((16,), jnp.float32)
        return 0

    lax.fori_loop(0, 8, _o, 0)

    pltpu.sync_copy(dst_hbm.at[wid], dstv)
    pltpu.sync_copy(zbuf, deg_sh.at[pl.ds(s * _RPT, _RPT)])
    plsc.subcore_barrier()

    def _chunk(j, _):
        pltpu.sync_copy(ones, deg_sh.at[dstv.at[j]], add=True)
        return 0

    lax.fori_loop(0, _NCH, _chunk, 0)
    plsc.subcore_barrier()
    pltpu.sync_copy(deg_sh.at[pl.ds(s * _RPT, _RPT)],
                    out_hbm.at[c, pl.ds(s * _RPT, _RPT)])


@functools.partial(
    pl.kernel,
    out_type=jax.ShapeDtypeStruct((_NC, _NPAD, _D), jnp.float32),
    mesh=_mesh,
    scratch_types=[
        pltpu.VMEM((_NCH, 128), jnp.int32),
        pltpu.VMEM((_NCH, 128), jnp.int32),
        pltpu.VMEM((128, _D), jnp.float32),
        pltpu.VMEM_SHARED((_NPAD, _D), jnp.float32),
    ],
)
def _sc_prop(hs_hbm, src_hbm, dst_hbm, out_hbm, srcv, dstv, buf, acc_sh):
    c = lax.axis_index("c")
    s = lax.axis_index("s")
    wid = c * _NS + s
    pltpu.sync_copy(src_hbm.at[wid], srcv)
    pltpu.sync_copy(dst_hbm.at[wid], dstv)
    pltpu.sync_copy(hs_hbm.at[pl.ds(s * _RPT, _RPT)],
                    acc_sh.at[pl.ds(s * _RPT, _RPT)])
    plsc.subcore_barrier()

    def _chunk(j, _):
        pltpu.sync_copy(hs_hbm.at[srcv.at[j]], buf)
        pltpu.sync_copy(buf, acc_sh.at[dstv.at[j]], add=True)
        return 0

    lax.fori_loop(0, _NCH, _chunk, 0)
    plsc.subcore_barrier()
    pltpu.sync_copy(acc_sh.at[pl.ds(s * _RPT, _RPT)],
                    out_hbm.at[c, pl.ds(s * _RPT, _RPT)])


def _lrelu(v):
    return jnp.where(v >= 0, v, 0.4 * v)


def _tc_in_body(x_ref, win_ref, bin_ref, wc1_ref, deg_ref, hs_ref, dinv_ref):
    i = pl.program_id(0)
    h0 = _lrelu(jnp.dot(x_ref[...], win_ref[...],
                        preferred_element_type=jnp.float32) + bin_ref[...])
    m1 = jnp.dot(h0, wc1_ref[...], preferred_element_type=jnp.float32)
    deg = deg_ref[0] + deg_ref[1] + 1.0
    row = i * _BLK + lax.broadcasted_iota(jnp.int32, (_BLK, 1), 0)
    dinv = jnp.where(row < _N, lax.rsqrt(deg), 0.0)
    dinv_ref[...] = dinv
    hs_ref[...] = dinv * m1


def _tc_mid_body(p_ref, hs1_ref, dinv_ref, bc1_ref, wc2_ref, hs2_ref):
    acc = p_ref[0] + p_ref[1] - hs1_ref[...]
    dinv = dinv_ref[...]
    h1 = _lrelu(dinv * acc + bc1_ref[...])
    hs2_ref[...] = dinv * jnp.dot(h1, wc2_ref[...],
                                  preferred_element_type=jnp.float32)


def _tc_out_body(p_ref, hs2_ref, dinv_ref, bc2_ref, batch_ref, wout_ref,
                 bout_ref, out_ref, s_acc, cnt_acc):
    i = pl.program_id(0)

    @pl.when(i == 0)
    def _():
        s_acc[...] = jnp.zeros_like(s_acc)
        cnt_acc[...] = jnp.zeros_like(cnt_acc)

    acc = p_ref[0] + p_ref[1] - hs2_ref[...]
    h2 = _lrelu(dinv_ref[...] * acc + bc2_ref[...])
    maskT = (batch_ref[...] ==
             lax.broadcasted_iota(jnp.int32, (_BLK, _G), 1)).astype(jnp.float32)
    dn = (((0,), (0,)), ((), ()))
    s_acc[...] += lax.dot_general(maskT, h2, dn,
                                  preferred_element_type=jnp.float32)
    cnt_acc[...] += lax.dot_general(maskT, jnp.ones((_BLK, _D), jnp.float32),
                                    dn, preferred_element_type=jnp.float32)

    @pl.when(i == _NBLK - 1)
    def _():
        pooled = s_acc[...] / jnp.maximum(cnt_acc[...], 1.0)
        out_ref[...] = jnp.dot(pooled, wout_ref[...],
                               preferred_element_type=jnp.float32) + bout_ref[...]


_tc_in = pl.pallas_call(
    _tc_in_body,
    grid=(_NBLK,),
    in_specs=[
        pl.BlockSpec((_BLK, _D), lambda i: (i, 0)),
        pl.BlockSpec((_D, _D), lambda i: (0, 0)),
        pl.BlockSpec((1, _D), lambda i: (0, 0)),
        pl.BlockSpec((_D, _D), lambda i: (0, 0)),
        pl.BlockSpec((_NC, _BLK, 1), lambda i: (0, i, 0)),
    ],
    out_specs=[
        pl.BlockSpec((_BLK, _D), lambda i: (i, 0)),
        pl.BlockSpec((_BLK, 1), lambda i: (i, 0)),
    ],
    out_shape=[
        jax.ShapeDtypeStruct((_NPAD, _D), jnp.float32),
        jax.ShapeDtypeStruct((_NPAD, 1), jnp.float32),
    ],
)

_tc_mid = pl.pallas_call(
    _tc_mid_body,
    grid=(_NBLK,),
    in_specs=[
        pl.BlockSpec((_NC, _BLK, _D), lambda i: (0, i, 0)),
        pl.BlockSpec((_BLK, _D), lambda i: (i, 0)),
        pl.BlockSpec((_BLK, 1), lambda i: (i, 0)),
        pl.BlockSpec((1, _D), lambda i: (0, 0)),
        pl.BlockSpec((_D, _D), lambda i: (0, 0)),
    ],
    out_specs=pl.BlockSpec((_BLK, _D), lambda i: (i, 0)),
    out_shape=jax.ShapeDtypeStruct((_NPAD, _D), jnp.float32),
)

_tc_out = pl.pallas_call(
    _tc_out_body,
    grid=(_NBLK,),
    in_specs=[
        pl.BlockSpec((_NC, _BLK, _D), lambda i: (0, i, 0)),
        pl.BlockSpec((_BLK, _D), lambda i: (i, 0)),
        pl.BlockSpec((_BLK, 1), lambda i: (i, 0)),
        pl.BlockSpec((1, _D), lambda i: (0, 0)),
        pl.BlockSpec((_BLK, 1), lambda i: (i, 0)),
        pl.BlockSpec((_D, _D), lambda i: (0, 0)),
        pl.BlockSpec((1, _D), lambda i: (0, 0)),
    ],
    out_specs=pl.BlockSpec((_G, _D), lambda i: (0, 0)),
    out_shape=jax.ShapeDtypeStruct((_G, _D), jnp.float32),
    scratch_shapes=[
        pltpu.VMEM((_G, _D), jnp.float32),
        pltpu.VMEM((_G, _D), jnp.float32),
    ],
)


def kernel(x, edge_index, batch, W_in, b_in, W_c1, b_c1, W_c2, b_c2, W_out, b_out):
    src = edge_index[0]
    dst = edge_index[1]
    epad = jnp.full((_EPAD - _E,), _N, jnp.int32)
    srcp = jnp.concatenate([src, epad]).reshape(_NW, _NCH, 128)
    dstp = jnp.concatenate([dst, epad]).reshape(_NW, _NCH, 128)
    x_p = jnp.pad(x, ((0, _NPAD - _N), (0, 0)))
    batch_p = jnp.concatenate(
        [batch, jnp.full((_NPAD - _N,), _G, jnp.int32)]).reshape(_NPAD, 1)

    deg = _sc_degree(dstp).reshape(_NC, _NPAD, 1)
    hs1, dinv = _tc_in(x_p, W_in, b_in.reshape(1, _D), W_c1, deg)
    p1 = _sc_prop(hs1, srcp, dstp)
    hs2 = _tc_mid(p1, hs1, dinv, b_c1.reshape(1, _D), W_c2)
    p2 = _sc_prop(hs2, srcp, dstp)
    return _tc_out(p2, hs2, dinv, b_c2.reshape(1, _D), batch_p,
                   W_out, b_out.reshape(1, _D))


# speedup vs baseline: 9.3388x; 1.1010x over previous
import functools

import jax
import jax.numpy as jnp
from jax import lax
from jax.experimental import pallas as pl
from jax.experimental.pallas import tpu as pltpu
from jax.experimental.pallas import tpu_sc as plsc

_N = 10000
_E = 320000
_D = 128
_G = 64
_NC = 2
_NS = 16
_NW = _NC * _NS
_NPAD = 10240
_RPT = _NPAD // _NS
_NCH = 80
_WCH = 40
_EPAD = _NCH * 128 * _NW
_BLK = 512
_NBLK = _NPAD // _BLK

_mesh = plsc.VectorSubcoreMesh(core_axis_name="c", subcore_axis_name="s")


@functools.partial(
    pl.kernel,
    out_type=jax.ShapeDtypeStruct((_NC, _NPAD), jnp.float32),
    mesh=_mesh,
    scratch_types=[
        pltpu.VMEM((_NCH, 128), jnp.int32),
        pltpu.VMEM((_RPT,), jnp.float32),
        pltpu.VMEM((128,), jnp.float32),
        pltpu.VMEM_SHARED((_NPAD,), jnp.float32),
        pltpu.SemaphoreType.DMA,
    ],
)
def _sc_degree(dst_hbm, out_hbm, dstv, zbuf, ones, deg_sh, dsem):
    c = lax.axis_index("c")
    s = lax.axis_index("s")
    wid = c * _NS + s

    def _z(i, _):
        zbuf[pl.ds(i * 16, 16)] = jnp.zeros((16,), jnp.float32)
        return 0

    lax.fori_loop(0, _RPT // 16, _z, 0)

    def _o(i, _):
        ones[pl.ds(i * 16, 16)] = jnp.ones((16,), jnp.float32)
        return 0

    lax.fori_loop(0, 8, _o, 0)

    pltpu.sync_copy(dst_hbm.at[wid], dstv)
    pltpu.sync_copy(zbuf, deg_sh.at[pl.ds(s * _RPT, _RPT)])
    plsc.subcore_barrier()

    def _fire(j, _):
        pltpu.async_copy(ones, deg_sh.at[dstv.at[j]], dsem, add=True)
        return 0

    lax.fori_loop(0, _NCH, _fire, 0)

    def _drain(j, _):
        pltpu.make_async_copy(ones, deg_sh.at[dstv.at[j]], dsem).wait()
        return 0

    lax.fori_loop(0, _NCH, _drain, 0)
    plsc.subcore_barrier()
    pltpu.sync_copy(deg_sh.at[pl.ds(s * _RPT, _RPT)],
                    out_hbm.at[c, pl.ds(s * _RPT, _RPT)])


@functools.partial(
    pl.kernel,
    out_type=jax.ShapeDtypeStruct((_NC, _NPAD, _D), jnp.float32),
    mesh=_mesh,
    scratch_types=[
        pltpu.VMEM((_WCH, 128), jnp.int32),
        pltpu.VMEM((_WCH, 128), jnp.int32),
        pltpu.VMEM((2, 128, _D), jnp.float32),
        pltpu.VMEM_SHARED((_NPAD, _D), jnp.float32),
        pltpu.SemaphoreType.DMA,
        pltpu.SemaphoreType.DMA,
        pltpu.SemaphoreType.DMA,
        pltpu.SemaphoreType.DMA,
    ],
)
def _sc_prop(hs_hbm, src_hbm, dst_hbm, out_hbm, srcw, dstw, buf, acc_sh,
             g0, g1, s0, s1):
    c = lax.axis_index("c")
    s = lax.axis_index("s")
    wid = c * _NS + s
    gs = (g0, g1)
    ss = (s0, s1)
    pltpu.sync_copy(hs_hbm.at[pl.ds(s * _RPT, _RPT)],
                    acc_sh.at[pl.ds(s * _RPT, _RPT)])
    plsc.subcore_barrier()

    def _gather(j, b, sem):
        return pltpu.async_copy(hs_hbm.at[srcw.at[j]], buf.at[b], sem)

    def _scatter(j, b, sem):
        return pltpu.async_copy(buf.at[b], acc_sh.at[dstw.at[j]], sem,
                                add=True)

    def _wait_g(j, b, sem):
        pltpu.make_async_copy(hs_hbm.at[srcw.at[j]], buf.at[b], sem).wait()

    def _wait_s(j, b, sem):
        pltpu.make_async_copy(buf.at[b], acc_sh.at[dstw.at[j]], sem).wait()

    for w in range(_NCH // _WCH):
        pltpu.sync_copy(src_hbm.at[wid, pl.ds(w * _WCH, _WCH)], srcw)
        pltpu.sync_copy(dst_hbm.at[wid, pl.ds(w * _WCH, _WCH)], dstw)
        _gather(0, 0, gs[0])
        _gather(1, 1, gs[1])
        _wait_g(0, 0, gs[0])
        _scatter(0, 0, ss[0])

        def _steady(a, _):
            for b in range(2):
                jl = 1 + 2 * a + b
                cur = (1 + b) % 2
                nxt = 1 - cur
                _wait_s(jl - 1, nxt, ss[nxt])
                _gather(jl + 1, nxt, gs[nxt])
                _wait_g(jl, cur, gs[cur])
                _scatter(jl, cur, ss[cur])
            return 0

        lax.fori_loop(0, (_WCH - 2) // 2, _steady, 0)
        _wait_g(_WCH - 1, 1, gs[1])
        _scatter(_WCH - 1, 1, ss[1])
        _wait_s(_WCH - 2, 0, ss[0])
        _wait_s(_WCH - 1, 1, ss[1])

    plsc.subcore_barrier()
    pltpu.sync_copy(acc_sh.at[pl.ds(s * _RPT, _RPT)],
                    out_hbm.at[c, pl.ds(s * _RPT, _RPT)])


def _lrelu(v):
    return jnp.where(v >= 0, v, 0.4 * v)


def _tc_in_body(x_ref, win_ref, bin_ref, wc1_ref, deg_ref, hs_ref, dinv_ref):
    i = pl.program_id(0)
    h0 = _lrelu(jnp.dot(x_ref[...], win_ref[...],
                        preferred_element_type=jnp.float32) + bin_ref[...])
    m1 = jnp.dot(h0, wc1_ref[...], preferred_element_type=jnp.float32)
    deg = deg_ref[0] + deg_ref[1] + 1.0
    row = i * _BLK + lax.broadcasted_iota(jnp.int32, (_BLK, 1), 0)
    dinv = jnp.where(row < _N, lax.rsqrt(deg), 0.0)
    dinv_ref[...] = dinv
    hs_ref[...] = dinv * m1


def _tc_mid_body(p_ref, hs1_ref, dinv_ref, bc1_ref, wc2_ref, hs2_ref):
    acc = p_ref[0] + p_ref[1] - hs1_ref[...]
    dinv = dinv_ref[...]
    h1 = _lrelu(dinv * acc + bc1_ref[...])
    hs2_ref[...] = dinv * jnp.dot(h1, wc2_ref[...],
                                  preferred_element_type=jnp.float32)


def _tc_out_body(p_ref, hs2_ref, dinv_ref, bc2_ref, batch_ref, wout_ref,
                 bout_ref, out_ref, s_acc, cnt_acc):
    i = pl.program_id(0)

    @pl.when(i == 0)
    def _():
        s_acc[...] = jnp.zeros_like(s_acc)
        cnt_acc[...] = jnp.zeros_like(cnt_acc)

    acc = p_ref[0] + p_ref[1] - hs2_ref[...]
    h2 = _lrelu(dinv_ref[...] * acc + bc2_ref[...])
    maskT = (batch_ref[...] ==
             lax.broadcasted_iota(jnp.int32, (_BLK, _G), 1)).astype(jnp.float32)
    dn = (((0,), (0,)), ((), ()))
    s_acc[...] += lax.dot_general(maskT, h2, dn,
                                  preferred_element_type=jnp.float32)
    cnt_acc[...] += lax.dot_general(maskT, jnp.ones((_BLK, _D), jnp.float32),
                                    dn, preferred_element_type=jnp.float32)

    @pl.when(i == _NBLK - 1)
    def _():
        pooled = s_acc[...] / jnp.maximum(cnt_acc[...], 1.0)
        out_ref[...] = jnp.dot(pooled, wout_ref[...],
                               preferred_element_type=jnp.float32) + bout_ref[...]


_tc_in = pl.pallas_call(
    _tc_in_body,
    grid=(_NBLK,),
    in_specs=[
        pl.BlockSpec((_BLK, _D), lambda i: (i, 0)),
        pl.BlockSpec((_D, _D), lambda i: (0, 0)),
        pl.BlockSpec((1, _D), lambda i: (0, 0)),
        pl.BlockSpec((_D, _D), lambda i: (0, 0)),
        pl.BlockSpec((_NC, _BLK, 1), lambda i: (0, i, 0)),
    ],
    out_specs=[
        pl.BlockSpec((_BLK, _D), lambda i: (i, 0)),
        pl.BlockSpec((_BLK, 1), lambda i: (i, 0)),
    ],
    out_shape=[
        jax.ShapeDtypeStruct((_NPAD, _D), jnp.float32),
        jax.ShapeDtypeStruct((_NPAD, 1), jnp.float32),
    ],
)

_tc_mid = pl.pallas_call(
    _tc_mid_body,
    grid=(_NBLK,),
    in_specs=[
        pl.BlockSpec((_NC, _BLK, _D), lambda i: (0, i, 0)),
        pl.BlockSpec((_BLK, _D), lambda i: (i, 0)),
        pl.BlockSpec((_BLK, 1), lambda i: (i, 0)),
        pl.BlockSpec((1, _D), lambda i: (0, 0)),
        pl.BlockSpec((_D, _D), lambda i: (0, 0)),
    ],
    out_specs=pl.BlockSpec((_BLK, _D), lambda i: (i, 0)),
    out_shape=jax.ShapeDtypeStruct((_NPAD, _D), jnp.float32),
)

_tc_out = pl.pallas_call(
    _tc_out_body,
    grid=(_NBLK,),
    in_specs=[
        pl.BlockSpec((_NC, _BLK, _D), lambda i: (0, i, 0)),
        pl.BlockSpec((_BLK, _D), lambda i: (i, 0)),
        pl.BlockSpec((_BLK, 1), lambda i: (i, 0)),
        pl.BlockSpec((1, _D), lambda i: (0, 0)),
        pl.BlockSpec((_BLK, 1), lambda i: (i, 0)),
        pl.BlockSpec((_D, _D), lambda i: (0, 0)),
        pl.BlockSpec((1, _D), lambda i: (0, 0)),
    ],
    out_specs=pl.BlockSpec((_G, _D), lambda i: (0, 0)),
    out_shape=jax.ShapeDtypeStruct((_G, _D), jnp.float32),
    scratch_shapes=[
        pltpu.VMEM((_G, _D), jnp.float32),
        pltpu.VMEM((_G, _D), jnp.float32),
    ],
)


def kernel(x, edge_index, batch, W_in, b_in, W_c1, b_c1, W_c2, b_c2, W_out, b_out):
    src = edge_index[0]
    dst = edge_index[1]
    epad = jnp.full((_EPAD - _E,), _N, jnp.int32)
    srcp = jnp.concatenate([src, epad]).reshape(_NW, _NCH, 128)
    dstp = jnp.concatenate([dst, epad]).reshape(_NW, _NCH, 128)
    x_p = jnp.pad(x, ((0, _NPAD - _N), (0, 0)))
    batch_p = jnp.concatenate(
        [batch, jnp.full((_NPAD - _N,), _G, jnp.int32)]).reshape(_NPAD, 1)

    deg = _sc_degree(dstp).reshape(_NC, _NPAD, 1)
    hs1, dinv = _tc_in(x_p, W_in, b_in.reshape(1, _D), W_c1, deg)
    p1 = _sc_prop(hs1, srcp, dstp)
    hs2 = _tc_mid(p1, hs1, dinv, b_c1.reshape(1, _D), W_c2)
    p2 = _sc_prop(hs2, srcp, dstp)
    return _tc_out(p2, hs2, dinv, b_c2.reshape(1, _D), batch_p,
                   W_out, b_out.reshape(1, _D))


# speedup vs baseline: 30.9168x; 3.3106x over previous
import functools

import jax
import jax.numpy as jnp
from jax import lax
from jax.experimental import pallas as pl
from jax.experimental.pallas import tpu as pltpu
from jax.experimental.pallas import tpu_sc as plsc

_N = 10000
_E = 320000
_D = 128
_G = 64
_NC = 2
_NS = 16
_NW = _NC * _NS
_NPAD = 10240
_RPT = _NPAD // _NS
_NCH = 80
_WCH = 40
_EPAD = _NCH * 128 * _NW
_BLK = 512
_NBLK = _NPAD // _BLK

_mesh = plsc.VectorSubcoreMesh(core_axis_name="c", subcore_axis_name="s")


@functools.partial(
    pl.kernel,
    out_type=jax.ShapeDtypeStruct((_NC, _NPAD), jnp.float32),
    mesh=_mesh,
    scratch_types=[
        pltpu.VMEM((_NCH, 128), jnp.int32),
        pltpu.VMEM((_RPT,), jnp.float32),
        pltpu.VMEM((128,), jnp.float32),
        pltpu.VMEM_SHARED((_NPAD,), jnp.float32),
        pltpu.SemaphoreType.DMA,
    ],
)
def _sc_degree(dst_hbm, out_hbm, dstv, zbuf, ones, deg_sh, dsem):
    c = lax.axis_index("c")
    s = lax.axis_index("s")
    wid = c * _NS + s

    def _z(i, _):
        zbuf[pl.ds(i * 16, 16)] = jnp.zeros((16,), jnp.float32)
        return 0

    lax.fori_loop(0, _RPT // 16, _z, 0)

    def _o(i, _):
        ones[pl.ds(i * 16, 16)] = jnp.ones((16,), jnp.float32)
        return 0

    lax.fori_loop(0, 8, _o, 0)

    pltpu.sync_copy(dst_hbm.at[wid], dstv)
    pltpu.sync_copy(zbuf, deg_sh.at[pl.ds(s * _RPT, _RPT)])
    plsc.subcore_barrier()

    def _fire(j, _):
        pltpu.async_copy(ones, deg_sh.at[dstv.at[j]], dsem, add=True)
        return 0

    lax.fori_loop(0, _NCH, _fire, 0)

    def _drain(j, _):
        pltpu.make_async_copy(ones, deg_sh.at[dstv.at[j]], dsem).wait()
        return 0

    lax.fori_loop(0, _NCH, _drain, 0)
    plsc.subcore_barrier()
    pltpu.sync_copy(deg_sh.at[pl.ds(s * _RPT, _RPT)],
                    out_hbm.at[c, pl.ds(s * _RPT, _RPT)])


@functools.partial(
    pl.kernel,
    out_type=jax.ShapeDtypeStruct((_NC, _NPAD, _D), jnp.float32),
    mesh=_mesh,
    scratch_types=[
        pltpu.VMEM((_WCH, 128), jnp.int32),
        pltpu.VMEM((_WCH, 128), jnp.int32),
        pltpu.VMEM((2, 128, _D), jnp.float32),
        pltpu.VMEM_SHARED((_NPAD, _D), jnp.float32),
        pltpu.SemaphoreType.DMA,
        pltpu.SemaphoreType.DMA,
        pltpu.SemaphoreType.DMA,
        pltpu.SemaphoreType.DMA,
    ],
)
def _sc_prop(hs_hbm, src_hbm, dst_hbm, out_hbm, srcw, dstw, buf, acc_sh,
             g0, g1, s0, s1):
    c = lax.axis_index("c")
    s = lax.axis_index("s")
    wid = c * _NS + s
    gs = (g0, g1)
    ss = (s0, s1)
    pltpu.sync_copy(hs_hbm.at[pl.ds(s * _RPT, _RPT)],
                    acc_sh.at[pl.ds(s * _RPT, _RPT)])
    plsc.subcore_barrier()

    def _gather(j, b, sem):
        return pltpu.async_copy(hs_hbm.at[srcw.at[j]], buf.at[b], sem)

    def _scatter(j, b, sem):
        return pltpu.async_copy(buf.at[b], acc_sh.at[dstw.at[j]], sem,
                                add=True)

    def _wait_g(j, b, sem):
        pltpu.make_async_copy(hs_hbm.at[srcw.at[j]], buf.at[b], sem).wait()

    def _wait_s(j, b, sem):
        pltpu.make_async_copy(buf.at[b], acc_sh.at[dstw.at[j]], sem).wait()

    for w in range(_NCH // _WCH):
        pltpu.sync_copy(src_hbm.at[wid, pl.ds(w * _WCH, _WCH)], srcw)
        pltpu.sync_copy(dst_hbm.at[wid, pl.ds(w * _WCH, _WCH)], dstw)
        _gather(0, 0, gs[0])
        _gather(1, 1, gs[1])
        _wait_g(0, 0, gs[0])
        _scatter(0, 0, ss[0])

        def _steady(a, _):
            for b in range(2):
                jl = 1 + 2 * a + b
                cur = (1 + b) % 2
                nxt = 1 - cur
                _wait_s(jl - 1, nxt, ss[nxt])
                _gather(jl + 1, nxt, gs[nxt])
                _wait_g(jl, cur, gs[cur])
                _scatter(jl, cur, ss[cur])
            return 0

        lax.fori_loop(0, (_WCH - 2) // 2, _steady, 0)
        _wait_g(_WCH - 1, 1, gs[1])
        _scatter(_WCH - 1, 1, ss[1])
        _wait_s(_WCH - 2, 0, ss[0])
        _wait_s(_WCH - 1, 1, ss[1])

    plsc.subcore_barrier()
    pltpu.sync_copy(acc_sh.at[pl.ds(s * _RPT, _RPT)],
                    out_hbm.at[c, pl.ds(s * _RPT, _RPT)])


def _lrelu(v):
    return jnp.where(v >= 0, v, 0.4 * v)


def _tc_in_body(x_ref, win_ref, bin_ref, wc1_ref, deg_ref, hs_ref, dinv_ref):
    i = pl.program_id(0)
    h0 = _lrelu(jnp.dot(x_ref[...], win_ref[...],
                        preferred_element_type=jnp.float32) + bin_ref[...])
    m1 = jnp.dot(h0, wc1_ref[...], preferred_element_type=jnp.float32)
    deg = deg_ref[0] + deg_ref[1] + 1.0
    row = i * _BLK + lax.broadcasted_iota(jnp.int32, (_BLK, 1), 0)
    dinv = jnp.where(row < _N, lax.rsqrt(deg), 0.0)
    dinv_ref[...] = dinv
    hs_ref[...] = dinv * m1


def _tc_mid_body(p_ref, hs1_ref, dinv_ref, bc1_ref, wc2_ref, hs2_ref):
    acc = p_ref[0] + p_ref[1] - hs1_ref[...]
    dinv = dinv_ref[...]
    h1 = _lrelu(dinv * acc + bc1_ref[...])
    hs2_ref[...] = dinv * jnp.dot(h1, wc2_ref[...],
                                  preferred_element_type=jnp.float32)


def _tc_out_body(p_ref, hs2_ref, dinv_ref, bc2_ref, batch_ref, wout_ref,
                 bout_ref, out_ref, s_acc, cnt_acc):
    i = pl.program_id(0)

    @pl.when(i == 0)
    def _():
        s_acc[...] = jnp.zeros_like(s_acc)
        cnt_acc[...] = jnp.zeros_like(cnt_acc)

    acc = p_ref[0] + p_ref[1] - hs2_ref[...]
    h2 = _lrelu(dinv_ref[...] * acc + bc2_ref[...])
    maskT = (batch_ref[...] ==
             lax.broadcasted_iota(jnp.int32, (_BLK, _G), 1)).astype(jnp.float32)
    dn = (((0,), (0,)), ((), ()))
    s_acc[...] += lax.dot_general(maskT, h2, dn,
                                  preferred_element_type=jnp.float32)
    cnt_acc[...] += lax.dot_general(maskT, jnp.ones((_BLK, _D), jnp.float32),
                                    dn, preferred_element_type=jnp.float32)

    @pl.when(i == _NBLK - 1)
    def _():
        pooled = s_acc[...] / jnp.maximum(cnt_acc[...], 1.0)
        out_ref[...] = jnp.dot(pooled, wout_ref[...],
                               preferred_element_type=jnp.float32) + bout_ref[...]


_tc_in = pl.pallas_call(
    _tc_in_body,
    grid=(_NBLK,),
    in_specs=[
        pl.BlockSpec((_BLK, _D), lambda i: (i, 0)),
        pl.BlockSpec((_D, _D), lambda i: (0, 0)),
        pl.BlockSpec((1, _D), lambda i: (0, 0)),
        pl.BlockSpec((_D, _D), lambda i: (0, 0)),
        pl.BlockSpec((_NC, _BLK, 1), lambda i: (0, i, 0)),
    ],
    out_specs=[
        pl.BlockSpec((_BLK, _D), lambda i: (i, 0)),
        pl.BlockSpec((_BLK, 1), lambda i: (i, 0)),
    ],
    out_shape=[
        jax.ShapeDtypeStruct((_NPAD, _D), jnp.float32),
        jax.ShapeDtypeStruct((_NPAD, 1), jnp.float32),
    ],
)

_tc_mid = pl.pallas_call(
    _tc_mid_body,
    grid=(_NBLK,),
    in_specs=[
        pl.BlockSpec((_NC, _BLK, _D), lambda i: (0, i, 0)),
        pl.BlockSpec((_BLK, _D), lambda i: (i, 0)),
        pl.BlockSpec((_BLK, 1), lambda i: (i, 0)),
        pl.BlockSpec((1, _D), lambda i: (0, 0)),
        pl.BlockSpec((_D, _D), lambda i: (0, 0)),
    ],
    out_specs=pl.BlockSpec((_BLK, _D), lambda i: (i, 0)),
    out_shape=jax.ShapeDtypeStruct((_NPAD, _D), jnp.float32),
)

_tc_out = pl.pallas_call(
    _tc_out_body,
    grid=(_NBLK,),
    in_specs=[
        pl.BlockSpec((_NC, _BLK, _D), lambda i: (0, i, 0)),
        pl.BlockSpec((_BLK, _D), lambda i: (i, 0)),
        pl.BlockSpec((_BLK, 1), lambda i: (i, 0)),
        pl.BlockSpec((1, _D), lambda i: (0, 0)),
        pl.BlockSpec((_BLK, 1), lambda i: (i, 0)),
        pl.BlockSpec((_D, _D), lambda i: (0, 0)),
        pl.BlockSpec((1, _D), lambda i: (0, 0)),
    ],
    out_specs=pl.BlockSpec((_G, _D), lambda i: (0, 0)),
    out_shape=jax.ShapeDtypeStruct((_G, _D), jnp.float32),
    scratch_shapes=[
        pltpu.VMEM((_G, _D), jnp.float32),
        pltpu.VMEM((_G, _D), jnp.float32),
    ],
)


def kernel(x, edge_index, batch, W_in, b_in, W_c1, b_c1, W_c2, b_c2, W_out, b_out):
    src = edge_index[0]
    dst = edge_index[1]
    epad = _N + jnp.arange(_EPAD - _E, dtype=jnp.int32) % (_NPAD - _N)
    srcp = jnp.concatenate([src, epad]).reshape(_NW, _NCH, 128)
    dstp = jnp.concatenate([dst, epad]).reshape(_NW, _NCH, 128)
    x_p = jnp.pad(x, ((0, _NPAD - _N), (0, 0)))
    batch_p = jnp.concatenate(
        [batch, jnp.full((_NPAD - _N,), _G, jnp.int32)]).reshape(_NPAD, 1)

    deg = _sc_degree(dstp).reshape(_NC, _NPAD, 1)
    hs1, dinv = _tc_in(x_p, W_in, b_in.reshape(1, _D), W_c1, deg)
    p1 = _sc_prop(hs1, srcp, dstp)
    hs2 = _tc_mid(p1, hs1, dinv, b_c1.reshape(1, _D), W_c2)
    p2 = _sc_prop(hs2, srcp, dstp)
    return _tc_out(p2, hs2, dinv, b_c2.reshape(1, _D), batch_p,
                   W_out, b_out.reshape(1, _D))


# speedup vs baseline: 32.1414x; 1.0396x over previous
import functools

import jax
import jax.numpy as jnp
from jax import lax
from jax.experimental import pallas as pl
from jax.experimental.pallas import tpu as pltpu
from jax.experimental.pallas import tpu_sc as plsc

_N = 10000
_E = 320000
_D = 128
_G = 64
_NC = 2
_NS = 16
_NW = _NC * _NS
_NPAD = 10240
_RPT = _NPAD // _NS
_NCH = 80
_WCH = 40
_EPAD = _NCH * 128 * _NW
_BLK = 512
_NBLK = _NPAD // _BLK

_mesh = plsc.VectorSubcoreMesh(core_axis_name="c", subcore_axis_name="s")


@functools.partial(
    pl.kernel,
    out_type=jax.ShapeDtypeStruct((_NC, _NPAD), jnp.float32),
    mesh=_mesh,
    scratch_types=[
        pltpu.VMEM((_NCH, 128), jnp.int32),
        pltpu.VMEM((_RPT,), jnp.float32),
        pltpu.VMEM((128,), jnp.float32),
        pltpu.VMEM_SHARED((_NPAD,), jnp.float32),
        pltpu.SemaphoreType.DMA,
    ],
)
def _sc_degree(dst_hbm, out_hbm, dstv, zbuf, ones, deg_sh, dsem):
    c = lax.axis_index("c")
    s = lax.axis_index("s")
    wid = c * _NS + s

    def _z(i, _):
        zbuf[pl.ds(i * 16, 16)] = jnp.zeros((16,), jnp.float32)
        return 0

    lax.fori_loop(0, _RPT // 16, _z, 0)

    def _o(i, _):
        ones[pl.ds(i * 16, 16)] = jnp.ones((16,), jnp.float32)
        return 0

    lax.fori_loop(0, 8, _o, 0)

    pltpu.sync_copy(dst_hbm.at[wid], dstv)
    pltpu.sync_copy(zbuf, deg_sh.at[pl.ds(s * _RPT, _RPT)])
    plsc.subcore_barrier()

    def _fire(j, _):
        pltpu.async_copy(ones, deg_sh.at[dstv.at[j]], dsem, add=True)
        return 0

    lax.fori_loop(0, _NCH, _fire, 0)

    def _drain(j, _):
        pltpu.make_async_copy(ones, deg_sh.at[dstv.at[j]], dsem).wait()
        return 0

    lax.fori_loop(0, _NCH, _drain, 0)
    plsc.subcore_barrier()
    pltpu.sync_copy(deg_sh.at[pl.ds(s * _RPT, _RPT)],
                    out_hbm.at[c, pl.ds(s * _RPT, _RPT)])


@functools.partial(
    pl.kernel,
    out_type=jax.ShapeDtypeStruct((_NC, _NPAD, _D), jnp.float32),
    mesh=_mesh,
    scratch_types=[
        pltpu.VMEM((_WCH, 128), jnp.int32),
        pltpu.VMEM((_WCH, 128), jnp.int32),
        pltpu.VMEM((2, 128, _D), jnp.float32),
        pltpu.VMEM_SHARED((_NPAD, _D), jnp.float32),
        pltpu.SemaphoreType.DMA,
        pltpu.SemaphoreType.DMA,
        pltpu.SemaphoreType.DMA,
        pltpu.SemaphoreType.DMA,
    ],
)
def _sc_prop(hs_hbm, src_hbm, dst_hbm, out_hbm, srcw, dstw, buf, acc_sh,
             g0, g1, s0, s1):
    c = lax.axis_index("c")
    s = lax.axis_index("s")
    wid = c * _NS + s
    gs = (g0, g1)
    ss = (s0, s1)
    pltpu.sync_copy(hs_hbm.at[pl.ds(s * _RPT, _RPT)],
                    acc_sh.at[pl.ds(s * _RPT, _RPT)])
    plsc.subcore_barrier()

    def _gather(j, b, sem):
        return pltpu.async_copy(hs_hbm.at[srcw.at[j]], buf.at[b], sem)

    def _scatter(j, b, sem):
        return pltpu.async_copy(buf.at[b], acc_sh.at[dstw.at[j]], sem,
                                add=True)

    def _wait_g(j, b, sem):
        pltpu.make_async_copy(hs_hbm.at[srcw.at[j]], buf.at[b], sem).wait()

    def _wait_s(j, b, sem):
        pltpu.make_async_copy(buf.at[b], acc_sh.at[dstw.at[j]], sem).wait()

    for w in range(_NCH // _WCH):
        pltpu.sync_copy(src_hbm.at[wid, pl.ds(w * _WCH, _WCH)], srcw)
        pltpu.sync_copy(dst_hbm.at[wid, pl.ds(w * _WCH, _WCH)], dstw)
        _gather(0, 0, gs[0])
        _gather(1, 1, gs[1])
        _wait_g(0, 0, gs[0])
        _scatter(0, 0, ss[0])

        def _steady(a, _):
            for b in range(2):
                jl = 1 + 2 * a + b
                cur = (1 + b) % 2
                nxt = 1 - cur
                _wait_s(jl - 1, nxt, ss[nxt])
                _gather(jl + 1, nxt, gs[nxt])
                _wait_g(jl, cur, gs[cur])
                _scatter(jl, cur, ss[cur])
            return 0

        lax.fori_loop(0, (_WCH - 2) // 2, _steady, 0)
        _wait_g(_WCH - 1, 1, gs[1])
        _scatter(_WCH - 1, 1, ss[1])
        _wait_s(_WCH - 2, 0, ss[0])
        _wait_s(_WCH - 1, 1, ss[1])

    plsc.subcore_barrier()
    pltpu.sync_copy(acc_sh.at[pl.ds(s * _RPT, _RPT)],
                    out_hbm.at[c, pl.ds(s * _RPT, _RPT)])


def _lrelu(v):
    return jnp.where(v >= 0, v, 0.4 * v)


def _dinv_col(deg_ref, i):
    deg = (deg_ref[0] + deg_ref[1] + 1.0).reshape(1, _BLK)
    dcol = jnp.transpose(deg, (1, 0))
    row = i * _BLK + lax.broadcasted_iota(jnp.int32, (_BLK, 1), 0)
    return jnp.where(row < _N, lax.rsqrt(dcol), 0.0)


def _tc_in_body(x_ref, win_ref, bin_ref, wc1_ref, deg_ref, hs_ref):
    i = pl.program_id(0)
    h0 = _lrelu(jnp.dot(x_ref[...], win_ref[...],
                        preferred_element_type=jnp.float32) + bin_ref[...])
    m1 = jnp.dot(h0, wc1_ref[...], preferred_element_type=jnp.float32)
    hs_ref[...] = _dinv_col(deg_ref, i) * m1


def _tc_mid_body(p_ref, hs1_ref, deg_ref, bc1_ref, wc2_ref, hs2_ref):
    dinv = _dinv_col(deg_ref, pl.program_id(0))
    acc = p_ref[0] + p_ref[1] - hs1_ref[...]
    h1 = _lrelu(dinv * acc + bc1_ref[...])
    hs2_ref[...] = dinv * jnp.dot(h1, wc2_ref[...],
                                  preferred_element_type=jnp.float32)


def _tc_out_body(p_ref, hs2_ref, deg_ref, bc2_ref, batch_ref, wout_ref,
                 bout_ref, out_ref, s_acc, cnt_acc):
    i = pl.program_id(0)

    @pl.when(i == 0)
    def _():
        s_acc[...] = jnp.zeros_like(s_acc)
        cnt_acc[...] = jnp.zeros_like(cnt_acc)

    acc = p_ref[0] + p_ref[1] - hs2_ref[...]
    h2 = _lrelu(_dinv_col(deg_ref, i) * acc + bc2_ref[...])
    mask = (batch_ref[...] ==
            lax.broadcasted_iota(jnp.int32, (_G, _BLK), 0)).astype(jnp.float32)
    s_acc[...] += jnp.dot(mask, h2, preferred_element_type=jnp.float32)
    cnt_acc[...] += jnp.dot(mask, jnp.ones((_BLK, _D), jnp.float32),
                            preferred_element_type=jnp.float32)

    @pl.when(i == _NBLK - 1)
    def _():
        pooled = s_acc[...] / jnp.maximum(cnt_acc[...], 1.0)
        out_ref[...] = jnp.dot(pooled, wout_ref[...],
                               preferred_element_type=jnp.float32) + bout_ref[...]


_tc_in = pl.pallas_call(
    _tc_in_body,
    grid=(_NBLK,),
    in_specs=[
        pl.BlockSpec((_BLK, _D), lambda i: (i, 0)),
        pl.BlockSpec((_D, _D), lambda i: (0, 0)),
        pl.BlockSpec((1, _D), lambda i: (0, 0)),
        pl.BlockSpec((_D, _D), lambda i: (0, 0)),
        pl.BlockSpec((_NC, _BLK), lambda i: (0, i)),
    ],
    out_specs=pl.BlockSpec((_BLK, _D), lambda i: (i, 0)),
    out_shape=jax.ShapeDtypeStruct((_NPAD, _D), jnp.float32),
)

_tc_mid = pl.pallas_call(
    _tc_mid_body,
    grid=(_NBLK,),
    in_specs=[
        pl.BlockSpec((_NC, _BLK, _D), lambda i: (0, i, 0)),
        pl.BlockSpec((_BLK, _D), lambda i: (i, 0)),
        pl.BlockSpec((_NC, _BLK), lambda i: (0, i)),
        pl.BlockSpec((1, _D), lambda i: (0, 0)),
        pl.BlockSpec((_D, _D), lambda i: (0, 0)),
    ],
    out_specs=pl.BlockSpec((_BLK, _D), lambda i: (i, 0)),
    out_shape=jax.ShapeDtypeStruct((_NPAD, _D), jnp.float32),
)

_tc_out = pl.pallas_call(
    _tc_out_body,
    grid=(_NBLK,),
    in_specs=[
        pl.BlockSpec((_NC, _BLK, _D), lambda i: (0, i, 0)),
        pl.BlockSpec((_BLK, _D), lambda i: (i, 0)),
        pl.BlockSpec((_NC, _BLK), lambda i: (0, i)),
        pl.BlockSpec((1, _D), lambda i: (0, 0)),
        pl.BlockSpec((1, _BLK), lambda i: (0, i)),
        pl.BlockSpec((_D, _D), lambda i: (0, 0)),
        pl.BlockSpec((1, _D), lambda i: (0, 0)),
    ],
    out_specs=pl.BlockSpec((_G, _D), lambda i: (0, 0)),
    out_shape=jax.ShapeDtypeStruct((_G, _D), jnp.float32),
    scratch_shapes=[
        pltpu.VMEM((_G, _D), jnp.float32),
        pltpu.VMEM((_G, _D), jnp.float32),
    ],
)


def kernel(x, edge_index, batch, W_in, b_in, W_c1, b_c1, W_c2, b_c2, W_out, b_out):
    src = edge_index[0]
    dst = edge_index[1]
    epad = _N + jnp.arange(_EPAD - _E, dtype=jnp.int32) % (_NPAD - _N)
    srcp = jnp.concatenate([src, epad]).reshape(_NW, _NCH, 128)
    dstp = jnp.concatenate([dst, epad]).reshape(_NW, _NCH, 128)
    batch_p = jnp.concatenate(
        [batch, jnp.full((_NPAD - _N,), _G, jnp.int32)]).reshape(1, _NPAD)

    deg = _sc_degree(dstp)
    hs1 = _tc_in(jnp.pad(x, ((0, _NPAD - _N), (0, 0))), W_in,
                 b_in.reshape(1, _D), W_c1, deg)
    p1 = _sc_prop(hs1, srcp, dstp)
    hs2 = _tc_mid(p1, hs1, deg, b_c1.reshape(1, _D), W_c2)
    p2 = _sc_prop(hs2, srcp, dstp)
    return _tc_out(p2, hs2, deg, b_c2.reshape(1, _D), batch_p,
                   W_out, b_out.reshape(1, _D))


# speedup vs baseline: 33.6114x; 1.0457x over previous
import functools

import jax
import jax.numpy as jnp
from jax import lax
from jax.experimental import pallas as pl
from jax.experimental.pallas import tpu as pltpu
from jax.experimental.pallas import tpu_sc as plsc

_N = 10000
_E = 320000
_D = 128
_G = 64
_NC = 2
_NS = 16
_NW = _NC * _NS
_NPAD = 10240
_RPT = _NPAD // _NS
_NCH = 80
_WCH = 40
_EPAD = _NCH * 128 * _NW
_BLK = 1024
_NBLK = _NPAD // _BLK

_mesh = plsc.VectorSubcoreMesh(core_axis_name="c", subcore_axis_name="s")


@functools.partial(
    pl.kernel,
    out_type=jax.ShapeDtypeStruct((_NC, _NPAD), jnp.float32),
    mesh=_mesh,
    scratch_types=[
        pltpu.VMEM((_NCH, 128), jnp.int32),
        pltpu.VMEM((_RPT,), jnp.float32),
        pltpu.VMEM((128,), jnp.float32),
        pltpu.VMEM_SHARED((_NPAD,), jnp.float32),
        pltpu.SemaphoreType.DMA,
    ],
)
def _sc_degree(dst_hbm, out_hbm, dstv, zbuf, ones, deg_sh, dsem):
    c = lax.axis_index("c")
    s = lax.axis_index("s")
    wid = c * _NS + s

    def _z(i, _):
        zbuf[pl.ds(i * 16, 16)] = jnp.zeros((16,), jnp.float32)
        return 0

    lax.fori_loop(0, _RPT // 16, _z, 0)

    def _o(i, _):
        ones[pl.ds(i * 16, 16)] = jnp.ones((16,), jnp.float32)
        return 0

    lax.fori_loop(0, 8, _o, 0)

    pltpu.sync_copy(dst_hbm.at[wid], dstv)
    pltpu.sync_copy(zbuf, deg_sh.at[pl.ds(s * _RPT, _RPT)])
    plsc.subcore_barrier()

    def _fire(j, _):
        pltpu.async_copy(ones, deg_sh.at[dstv.at[j]], dsem, add=True)
        return 0

    lax.fori_loop(0, _NCH, _fire, 0)

    def _drain(j, _):
        pltpu.make_async_copy(ones, deg_sh.at[dstv.at[j]], dsem).wait()
        return 0

    lax.fori_loop(0, _NCH, _drain, 0)
    plsc.subcore_barrier()
    pltpu.sync_copy(deg_sh.at[pl.ds(s * _RPT, _RPT)],
                    out_hbm.at[c, pl.ds(s * _RPT, _RPT)])


@functools.partial(
    pl.kernel,
    out_type=jax.ShapeDtypeStruct((_NC, _NPAD, _D), jnp.float32),
    mesh=_mesh,
    scratch_types=[
        pltpu.VMEM((_WCH, 128), jnp.int32),
        pltpu.VMEM((_WCH, 128), jnp.int32),
        pltpu.VMEM((2, 128, _D), jnp.float32),
        pltpu.VMEM_SHARED((_NPAD, _D), jnp.float32),
        pltpu.SemaphoreType.DMA,
        pltpu.SemaphoreType.DMA,
        pltpu.SemaphoreType.DMA,
        pltpu.SemaphoreType.DMA,
    ],
)
def _sc_prop(hs_hbm, src_hbm, dst_hbm, out_hbm, srcw, dstw, buf, acc_sh,
             g0, g1, s0, s1):
    c = lax.axis_index("c")
    s = lax.axis_index("s")
    wid = c * _NS + s
    gs = (g0, g1)
    ss = (s0, s1)
    pltpu.sync_copy(hs_hbm.at[pl.ds(s * _RPT, _RPT)],
                    acc_sh.at[pl.ds(s * _RPT, _RPT)])
    plsc.subcore_barrier()

    def _gather(j, b, sem):
        return pltpu.async_copy(hs_hbm.at[srcw.at[j]], buf.at[b], sem)

    def _scatter(j, b, sem):
        return pltpu.async_copy(buf.at[b], acc_sh.at[dstw.at[j]], sem,
                                add=True)

    def _wait_g(j, b, sem):
        pltpu.make_async_copy(hs_hbm.at[srcw.at[j]], buf.at[b], sem).wait()

    def _wait_s(j, b, sem):
        pltpu.make_async_copy(buf.at[b], acc_sh.at[dstw.at[j]], sem).wait()

    for w in range(_NCH // _WCH):
        pltpu.sync_copy(src_hbm.at[wid, pl.ds(w * _WCH, _WCH)], srcw)
        pltpu.sync_copy(dst_hbm.at[wid, pl.ds(w * _WCH, _WCH)], dstw)
        _gather(0, 0, gs[0])
        _gather(1, 1, gs[1])
        _wait_g(0, 0, gs[0])
        _scatter(0, 0, ss[0])

        def _steady(a, _):
            for b in range(2):
                jl = 1 + 2 * a + b
                cur = (1 + b) % 2
                nxt = 1 - cur
                _wait_s(jl - 1, nxt, ss[nxt])
                _gather(jl + 1, nxt, gs[nxt])
                _wait_g(jl, cur, gs[cur])
                _scatter(jl, cur, ss[cur])
            return 0

        lax.fori_loop(0, (_WCH - 2) // 2, _steady, 0)
        _wait_g(_WCH - 1, 1, gs[1])
        _scatter(_WCH - 1, 1, ss[1])
        _wait_s(_WCH - 2, 0, ss[0])
        _wait_s(_WCH - 1, 1, ss[1])

    plsc.subcore_barrier()
    pltpu.sync_copy(acc_sh.at[pl.ds(s * _RPT, _RPT)],
                    out_hbm.at[c, pl.ds(s * _RPT, _RPT)])


def _lrelu(v):
    return jnp.where(v >= 0, v, 0.4 * v)


def _dinv_col(deg_ref, i):
    deg = (deg_ref[0] + deg_ref[1] + 1.0).reshape(1, _BLK)
    dcol = jnp.transpose(deg, (1, 0))
    row = i * _BLK + lax.broadcasted_iota(jnp.int32, (_BLK, 1), 0)
    return jnp.where(row < _N, lax.rsqrt(dcol), 0.0)


def _tc_mm_body(x_ref, win_ref, bin_ref, wc1_ref, m1_ref):
    h0 = _lrelu(jnp.dot(x_ref[...], win_ref[...],
                        preferred_element_type=jnp.float32) + bin_ref[...])
    m1_ref[...] = jnp.dot(h0, wc1_ref[...], preferred_element_type=jnp.float32)


def _tc_scale_body(m1_ref, deg_ref, hs_ref):
    i = pl.program_id(0)
    row = i * _BLK + lax.broadcasted_iota(jnp.int32, (_BLK, 1), 0)
    hs_ref[...] = jnp.where(row < _N, _dinv_col(deg_ref, i) * m1_ref[...], 0.0)


def _tc_mid_body(p_ref, hs1_ref, deg_ref, bc1_ref, wc2_ref, hs2_ref):
    dinv = _dinv_col(deg_ref, pl.program_id(0))
    acc = p_ref[0] + p_ref[1] - hs1_ref[...]
    h1 = _lrelu(dinv * acc + bc1_ref[...])
    hs2_ref[...] = dinv * jnp.dot(h1, wc2_ref[...],
                                  preferred_element_type=jnp.float32)


def _tc_out_body(p_ref, hs2_ref, deg_ref, bc2_ref, batch_ref, wout_ref,
                 bout_ref, out_ref, s_acc, cnt_acc):
    i = pl.program_id(0)

    @pl.when(i == 0)
    def _():
        s_acc[...] = jnp.zeros_like(s_acc)
        cnt_acc[...] = jnp.zeros_like(cnt_acc)

    acc = p_ref[0] + p_ref[1] - hs2_ref[...]
    h2 = _lrelu(_dinv_col(deg_ref, i) * acc + bc2_ref[...])
    mask = (batch_ref[...] ==
            lax.broadcasted_iota(jnp.int32, (_G, _BLK), 0)).astype(jnp.float32)
    s_acc[...] += jnp.dot(mask, h2, preferred_element_type=jnp.float32)
    cnt_acc[...] += jnp.dot(mask, jnp.ones((_BLK, _D), jnp.float32),
                            preferred_element_type=jnp.float32)

    @pl.when(i == _NBLK - 1)
    def _():
        pooled = s_acc[...] / jnp.maximum(cnt_acc[...], 1.0)
        out_ref[...] = jnp.dot(pooled, wout_ref[...],
                               preferred_element_type=jnp.float32) + bout_ref[...]


_tc_mm = pl.pallas_call(
    _tc_mm_body,
    grid=(_NBLK,),
    in_specs=[
        pl.BlockSpec((_BLK, _D), lambda i: (i, 0)),
        pl.BlockSpec((_D, _D), lambda i: (0, 0)),
        pl.BlockSpec((1, _D), lambda i: (0, 0)),
        pl.BlockSpec((_D, _D), lambda i: (0, 0)),
    ],
    out_specs=pl.BlockSpec((_BLK, _D), lambda i: (i, 0)),
    out_shape=jax.ShapeDtypeStruct((_NPAD, _D), jnp.float32),
)

_tc_scale = pl.pallas_call(
    _tc_scale_body,
    grid=(_NBLK,),
    in_specs=[
        pl.BlockSpec((_BLK, _D), lambda i: (i, 0)),
        pl.BlockSpec((_NC, _BLK), lambda i: (0, i)),
    ],
    out_specs=pl.BlockSpec((_BLK, _D), lambda i: (i, 0)),
    out_shape=jax.ShapeDtypeStruct((_NPAD, _D), jnp.float32),
)

_tc_mid = pl.pallas_call(
    _tc_mid_body,
    grid=(_NBLK,),
    in_specs=[
        pl.BlockSpec((_NC, _BLK, _D), lambda i: (0, i, 0)),
        pl.BlockSpec((_BLK, _D), lambda i: (i, 0)),
        pl.BlockSpec((_NC, _BLK), lambda i: (0, i)),
        pl.BlockSpec((1, _D), lambda i: (0, 0)),
        pl.BlockSpec((_D, _D), lambda i: (0, 0)),
    ],
    out_specs=pl.BlockSpec((_BLK, _D), lambda i: (i, 0)),
    out_shape=jax.ShapeDtypeStruct((_NPAD, _D), jnp.float32),
)

_tc_out = pl.pallas_call(
    _tc_out_body,
    grid=(_NBLK,),
    in_specs=[
        pl.BlockSpec((_NC, _BLK, _D), lambda i: (0, i, 0)),
        pl.BlockSpec((_BLK, _D), lambda i: (i, 0)),
        pl.BlockSpec((_NC, _BLK), lambda i: (0, i)),
        pl.BlockSpec((1, _D), lambda i: (0, 0)),
        pl.BlockSpec((1, _BLK), lambda i: (0, i)),
        pl.BlockSpec((_D, _D), lambda i: (0, 0)),
        pl.BlockSpec((1, _D), lambda i: (0, 0)),
    ],
    out_specs=pl.BlockSpec((_G, _D), lambda i: (0, 0)),
    out_shape=jax.ShapeDtypeStruct((_G, _D), jnp.float32),
    scratch_shapes=[
        pltpu.VMEM((_G, _D), jnp.float32),
        pltpu.VMEM((_G, _D), jnp.float32),
    ],
)


def kernel(x, edge_index, batch, W_in, b_in, W_c1, b_c1, W_c2, b_c2, W_out, b_out):
    src = edge_index[0]
    dst = edge_index[1]
    epad = _N + jnp.arange(_EPAD - _E, dtype=jnp.int32) % (_NPAD - _N)
    srcp = jnp.concatenate([src, epad]).reshape(_NW, _NCH, 128)
    dstp = jnp.concatenate([dst, epad]).reshape(_NW, _NCH, 128)
    batch_p = jnp.concatenate(
        [batch, jnp.full((_NPAD - _N,), _G, jnp.int32)]).reshape(1, _NPAD)

    deg = _sc_degree(dstp)
    m1 = _tc_mm(x, W_in, b_in.reshape(1, _D), W_c1)
    hs1 = _tc_scale(m1, deg)
    p1 = _sc_prop(hs1, srcp, dstp)
    hs2 = _tc_mid(p1, hs1, deg, b_c1.reshape(1, _D), W_c2)
    p2 = _sc_prop(hs2, srcp, dstp)
    return _tc_out(p2, hs2, deg, b_c2.reshape(1, _D), batch_p,
                   W_out, b_out.reshape(1, _D))


# speedup vs baseline: 34.3609x; 1.0223x over previous
import functools

import jax
import jax.numpy as jnp
from jax import lax
from jax.experimental import pallas as pl
from jax.experimental.pallas import tpu as pltpu
from jax.experimental.pallas import tpu_sc as plsc

_N = 10000
_E = 320000
_D = 128
_G = 64
_NC = 2
_NS = 16
_NW = _NC * _NS
_NPAD = 10240
_RPT = _NPAD // _NS
_NCH = 80
_WCH = 40
_NCHP = _NCH * _NW
_BLK = 1024
_NBLK = _NPAD // _BLK

_mesh = plsc.VectorSubcoreMesh(core_axis_name="c", subcore_axis_name="s")


@functools.partial(
    pl.kernel,
    out_type=jax.ShapeDtypeStruct((_NC, _NPAD), jnp.float32),
    mesh=_mesh,
    scratch_types=[
        pltpu.VMEM((_NCH, 128), jnp.int32),
        pltpu.VMEM((_RPT,), jnp.float32),
        pltpu.VMEM((128,), jnp.float32),
        pltpu.VMEM_SHARED((_NPAD,), jnp.float32),
        pltpu.SemaphoreType.DMA,
    ],
)
def _sc_degree(ei_hbm, out_hbm, dstv, zbuf, ones, deg_sh, dsem):
    c = lax.axis_index("c")
    s = lax.axis_index("s")
    wid = c * _NS + s
    base = pl.multiple_of(wid * _NCH, 8)

    def _z(i, _):
        zbuf[pl.ds(i * 16, 16)] = jnp.zeros((16,), jnp.float32)
        return 0

    lax.fori_loop(0, _RPT // 16, _z, 0)

    def _o(i, _):
        ones[pl.ds(i * 16, 16)] = jnp.ones((16,), jnp.float32)
        return 0

    lax.fori_loop(0, 8, _o, 0)

    pltpu.sync_copy(ei_hbm.at[1, pl.ds(base, _NCH)], dstv)
    pltpu.sync_copy(zbuf, deg_sh.at[pl.ds(s * _RPT, _RPT)])
    plsc.subcore_barrier()

    def _fire(j, _):
        pltpu.async_copy(ones, deg_sh.at[dstv.at[j]], dsem, add=True)
        return 0

    lax.fori_loop(0, _NCH, _fire, 0)

    def _drain(j, _):
        pltpu.make_async_copy(ones, deg_sh.at[dstv.at[j]], dsem).wait()
        return 0

    lax.fori_loop(0, _NCH, _drain, 0)
    plsc.subcore_barrier()
    pltpu.sync_copy(deg_sh.at[pl.ds(s * _RPT, _RPT)],
                    out_hbm.at[c, pl.ds(s * _RPT, _RPT)])


@functools.partial(
    pl.kernel,
    out_type=jax.ShapeDtypeStruct((_NC, _NPAD, _D), jnp.float32),
    mesh=_mesh,
    scratch_types=[
        pltpu.VMEM((_WCH, 128), jnp.int32),
        pltpu.VMEM((_WCH, 128), jnp.int32),
        pltpu.VMEM((2, 128, _D), jnp.float32),
        pltpu.VMEM_SHARED((_NPAD, _D), jnp.float32),
        pltpu.SemaphoreType.DMA,
        pltpu.SemaphoreType.DMA,
        pltpu.SemaphoreType.DMA,
        pltpu.SemaphoreType.DMA,
    ],
)
def _sc_prop(hs_hbm, ei_hbm, out_hbm, srcw, dstw, buf, acc_sh,
             g0, g1, s0, s1):
    c = lax.axis_index("c")
    s = lax.axis_index("s")
    wid = c * _NS + s
    gs = (g0, g1)
    ss = (s0, s1)
    pltpu.sync_copy(hs_hbm.at[pl.ds(s * _RPT, _RPT)],
                    acc_sh.at[pl.ds(s * _RPT, _RPT)])
    plsc.subcore_barrier()

    def _gather(j, b, sem):
        return pltpu.async_copy(hs_hbm.at[srcw.at[j]], buf.at[b], sem)

    def _scatter(j, b, sem):
        return pltpu.async_copy(buf.at[b], acc_sh.at[dstw.at[j]], sem,
                                add=True)

    def _wait_g(j, b, sem):
        pltpu.make_async_copy(hs_hbm.at[srcw.at[j]], buf.at[b], sem).wait()

    def _wait_s(j, b, sem):
        pltpu.make_async_copy(buf.at[b], acc_sh.at[dstw.at[j]], sem).wait()

    def _win(load_base, ln):
        off = pl.multiple_of(load_base, 8)
        pltpu.sync_copy(ei_hbm.at[0, pl.ds(off, _WCH)], srcw)
        pltpu.sync_copy(ei_hbm.at[1, pl.ds(off, _WCH)], dstw)
        _gather(0, 0, gs[0])
        _gather(1, 1, gs[1])
        _wait_g(0, 0, gs[0])
        _scatter(0, 0, ss[0])

        def _steady(a, _):
            for b in range(2):
                jl = 1 + 2 * a + b
                cur = (1 + b) % 2
                nxt = 1 - cur
                _wait_s(jl - 1, nxt, ss[nxt])
                _gather(jl + 1, nxt, gs[nxt])
                _wait_g(jl, cur, gs[cur])
                _scatter(jl, cur, ss[cur])
            return 0

        lax.fori_loop(0, (ln - 2) // 2, _steady, 0)
        _wait_g(ln - 1, 1, gs[1])
        _scatter(ln - 1, 1, ss[1])
        _wait_s(ln - 2, 0, ss[0])
        _wait_s(ln - 1, 1, ss[1])

    _win(wid * _NCH, _WCH)
    _win(wid * _NCH + _WCH, _WCH)

    plsc.subcore_barrier()
    pltpu.sync_copy(acc_sh.at[pl.ds(s * _RPT, _RPT)],
                    out_hbm.at[c, pl.ds(s * _RPT, _RPT)])


def _lrelu(v):
    return jnp.where(v >= 0, v, 0.4 * v)


def _dinv_col(deg_ref, i):
    deg = (deg_ref[0] + deg_ref[1] + 1.0).reshape(1, _BLK)
    dcol = jnp.transpose(deg, (1, 0))
    row = i * _BLK + lax.broadcasted_iota(jnp.int32, (_BLK, 1), 0)
    return jnp.where(row < _N, lax.rsqrt(dcol), 0.0)


def _tc_mm_body(x_ref, win_ref, bin_ref, wc1_ref, m1_ref):
    h0 = _lrelu(jnp.dot(x_ref[...], win_ref[...],
                        preferred_element_type=jnp.float32) + bin_ref[...])
    m1_ref[...] = jnp.dot(h0, wc1_ref[...], preferred_element_type=jnp.float32)


def _tc_scale_body(m1_ref, deg_ref, hs_ref):
    i = pl.program_id(0)
    row = i * _BLK + lax.broadcasted_iota(jnp.int32, (_BLK, 1), 0)
    hs_ref[...] = jnp.where(row < _N, _dinv_col(deg_ref, i) * m1_ref[...], 0.0)


def _tc_mid_body(p_ref, hs1_ref, deg_ref, bc1_ref, wc2_ref, hs2_ref):
    dinv = _dinv_col(deg_ref, pl.program_id(0))
    acc = p_ref[0] + p_ref[1] - hs1_ref[...]
    h1 = _lrelu(dinv * acc + bc1_ref[...])
    hs2_ref[...] = dinv * jnp.dot(h1, wc2_ref[...],
                                  preferred_element_type=jnp.float32)


def _tc_out_body(p_ref, hs2_ref, deg_ref, bc2_ref, batch_ref, wout_ref,
                 bout_ref, out_ref, s_acc, cnt_acc):
    i = pl.program_id(0)

    @pl.when(i == 0)
    def _():
        s_acc[...] = jnp.zeros_like(s_acc)
        cnt_acc[...] = jnp.zeros_like(cnt_acc)

    acc = p_ref[0] + p_ref[1] - hs2_ref[...]
    h2 = _lrelu(_dinv_col(deg_ref, i) * acc + bc2_ref[...])
    n_col = i * _BLK + lax.broadcasted_iota(jnp.int32, (_G, _BLK), 1)
    mask = ((batch_ref[...] == lax.broadcasted_iota(jnp.int32, (_G, _BLK), 0))
            & (n_col < _N)).astype(jnp.float32)
    s_acc[...] += jnp.dot(mask, h2, preferred_element_type=jnp.float32)
    cnt_acc[...] += jnp.dot(mask, jnp.ones((_BLK, _D), jnp.float32),
                            preferred_element_type=jnp.float32)

    @pl.when(i == _NBLK - 1)
    def _():
        pooled = s_acc[...] / jnp.maximum(cnt_acc[...], 1.0)
        out_ref[...] = jnp.dot(pooled, wout_ref[...],
                               preferred_element_type=jnp.float32) + bout_ref[...]


_tc_mm = pl.pallas_call(
    _tc_mm_body,
    grid=(_NBLK,),
    in_specs=[
        pl.BlockSpec((_BLK, _D), lambda i: (i, 0)),
        pl.BlockSpec((_D, _D), lambda i: (0, 0)),
        pl.BlockSpec((1, _D), lambda i: (0, 0)),
        pl.BlockSpec((_D, _D), lambda i: (0, 0)),
    ],
    out_specs=pl.BlockSpec((_BLK, _D), lambda i: (i, 0)),
    out_shape=jax.ShapeDtypeStruct((_NPAD, _D), jnp.float32),
)

_tc_scale = pl.pallas_call(
    _tc_scale_body,
    grid=(_NBLK,),
    in_specs=[
        pl.BlockSpec((_BLK, _D), lambda i: (i, 0)),
        pl.BlockSpec((_NC, _BLK), lambda i: (0, i)),
    ],
    out_specs=pl.BlockSpec((_BLK, _D), lambda i: (i, 0)),
    out_shape=jax.ShapeDtypeStruct((_NPAD, _D), jnp.float32),
)

_tc_mid = pl.pallas_call(
    _tc_mid_body,
    grid=(_NBLK,),
    in_specs=[
        pl.BlockSpec((_NC, _BLK, _D), lambda i: (0, i, 0)),
        pl.BlockSpec((_BLK, _D), lambda i: (i, 0)),
        pl.BlockSpec((_NC, _BLK), lambda i: (0, i)),
        pl.BlockSpec((1, _D), lambda i: (0, 0)),
        pl.BlockSpec((_D, _D), lambda i: (0, 0)),
    ],
    out_specs=pl.BlockSpec((_BLK, _D), lambda i: (i, 0)),
    out_shape=jax.ShapeDtypeStruct((_NPAD, _D), jnp.float32),
)

_tc_out = pl.pallas_call(
    _tc_out_body,
    grid=(_NBLK,),
    in_specs=[
        pl.BlockSpec((_NC, _BLK, _D), lambda i: (0, i, 0)),
        pl.BlockSpec((_BLK, _D), lambda i: (i, 0)),
        pl.BlockSpec((_NC, _BLK), lambda i: (0, i)),
        pl.BlockSpec((1, _D), lambda i: (0, 0)),
        pl.BlockSpec((1, _BLK), lambda i: (0, i)),
        pl.BlockSpec((_D, _D), lambda i: (0, 0)),
        pl.BlockSpec((1, _D), lambda i: (0, 0)),
    ],
    out_specs=pl.BlockSpec((_G, _D), lambda i: (0, 0)),
    out_shape=jax.ShapeDtypeStruct((_G, _D), jnp.float32),
    scratch_shapes=[
        pltpu.VMEM((_G, _D), jnp.float32),
        pltpu.VMEM((_G, _D), jnp.float32),
    ],
)


def kernel(x, edge_index, batch, W_in, b_in, W_c1, b_c1, W_c2, b_c2, W_out, b_out):
    cyc = _N + jnp.arange(_NCHP * 128 - _E, dtype=jnp.int32) % (_NPAD - _N)
    ei3 = jnp.concatenate(
        [edge_index, jnp.broadcast_to(cyc, (2, cyc.shape[0]))],
        axis=1).reshape(2, _NCHP, 128)
    batch_r = batch.reshape(1, _N)

    deg = _sc_degree(ei3)
    m1 = _tc_mm(x, W_in, b_in.reshape(1, _D), W_c1)
    hs1 = _tc_scale(m1, deg)
    p1 = _sc_prop(hs1, ei3)
    hs2 = _tc_mid(p1, hs1, deg, b_c1.reshape(1, _D), W_c2)
    p2 = _sc_prop(hs2, ei3)
    return _tc_out(p2, hs2, deg, b_c2.reshape(1, _D), batch_r,
                   W_out, b_out.reshape(1, _D))


# speedup vs baseline: 35.9402x; 1.0460x over previous
import functools

import jax
import jax.numpy as jnp
from jax import lax
from jax.experimental import pallas as pl
from jax.experimental.pallas import tpu as pltpu
from jax.experimental.pallas import tpu_sc as plsc

_N = 10000
_E = 320000
_D = 128
_G = 64
_NC = 2
_NS = 16
_NW = _NC * _NS
_NPAD = 10240
_RPT = _NPAD // _NS
_NCH = 80
_WCH = 40
_NCHP = _NCH * _NW
_BLK = 2048
_NBLK = _NPAD // _BLK

_mesh = plsc.VectorSubcoreMesh(core_axis_name="c", subcore_axis_name="s")


@functools.partial(
    pl.kernel,
    out_type=jax.ShapeDtypeStruct((_NC, _NPAD), jnp.float32),
    mesh=_mesh,
    scratch_types=[
        pltpu.VMEM((_NCH, 128), jnp.int32),
        pltpu.VMEM((_RPT,), jnp.float32),
        pltpu.VMEM((128,), jnp.float32),
        pltpu.VMEM_SHARED((_NPAD,), jnp.float32),
        pltpu.SemaphoreType.DMA,
    ],
)
def _sc_degree(ei_hbm, out_hbm, dstv, zbuf, ones, deg_sh, dsem):
    c = lax.axis_index("c")
    s = lax.axis_index("s")
    wid = c * _NS + s
    base = pl.multiple_of(wid * _NCH, 8)

    def _z(i, _):
        zbuf[pl.ds(i * 16, 16)] = jnp.zeros((16,), jnp.float32)
        return 0

    lax.fori_loop(0, _RPT // 16, _z, 0)

    def _o(i, _):
        ones[pl.ds(i * 16, 16)] = jnp.ones((16,), jnp.float32)
        return 0

    lax.fori_loop(0, 8, _o, 0)

    pltpu.sync_copy(ei_hbm.at[1, pl.ds(base, _NCH)], dstv)
    pltpu.sync_copy(zbuf, deg_sh.at[pl.ds(s * _RPT, _RPT)])
    plsc.subcore_barrier()

    def _fire(j, _):
        pltpu.async_copy(ones, deg_sh.at[dstv.at[j]], dsem, add=True)
        return 0

    lax.fori_loop(0, _NCH, _fire, 0)

    def _drain(j, _):
        pltpu.make_async_copy(ones, deg_sh.at[dstv.at[j]], dsem).wait()
        return 0

    lax.fori_loop(0, _NCH, _drain, 0)
    plsc.subcore_barrier()
    pltpu.sync_copy(deg_sh.at[pl.ds(s * _RPT, _RPT)],
                    out_hbm.at[c, pl.ds(s * _RPT, _RPT)])


@functools.partial(
    pl.kernel,
    out_type=jax.ShapeDtypeStruct((_NC, _NPAD, _D), jnp.float32),
    mesh=_mesh,
    scratch_types=[
        pltpu.VMEM((_WCH, 128), jnp.int32),
        pltpu.VMEM((_WCH, 128), jnp.int32),
        pltpu.VMEM((2, 128, _D), jnp.float32),
        pltpu.VMEM_SHARED((_NPAD, _D), jnp.float32),
        pltpu.SemaphoreType.DMA,
        pltpu.SemaphoreType.DMA,
        pltpu.SemaphoreType.DMA,
        pltpu.SemaphoreType.DMA,
        pltpu.SemaphoreType.DMA,
    ],
)
def _sc_prop(hs_hbm, ei_hbm, out_hbm, srcw, dstw, buf, acc_sh,
             g0, g1, s0, s1, isem):
    c = lax.axis_index("c")
    s = lax.axis_index("s")
    wid = c * _NS + s
    gs = (g0, g1)
    ss = (s0, s1)
    pltpu.async_copy(hs_hbm.at[pl.ds(s * _RPT, _RPT)],
                     acc_sh.at[pl.ds(s * _RPT, _RPT)], isem)

    def _gather(j, b, sem):
        return pltpu.async_copy(hs_hbm.at[srcw.at[j]], buf.at[b], sem)

    def _scatter(j, b, sem):
        return pltpu.async_copy(buf.at[b], acc_sh.at[dstw.at[j]], sem,
                                add=True)

    def _wait_g(j, b, sem):
        pltpu.make_async_copy(hs_hbm.at[srcw.at[j]], buf.at[b], sem).wait()

    def _wait_s(j, b, sem):
        pltpu.make_async_copy(buf.at[b], acc_sh.at[dstw.at[j]], sem).wait()

    def _load_win(load_base):
        off = pl.multiple_of(load_base, 8)
        pltpu.sync_copy(ei_hbm.at[0, pl.ds(off, _WCH)], srcw)
        pltpu.sync_copy(ei_hbm.at[1, pl.ds(off, _WCH)], dstw)

    def _win(load_base, ln, loaded=False):
        if not loaded:
            _load_win(load_base)
        _gather(0, 0, gs[0])
        _gather(1, 1, gs[1])
        _wait_g(0, 0, gs[0])
        _scatter(0, 0, ss[0])

        def _steady(a, _):
            for b in range(2):
                jl = 1 + 2 * a + b
                cur = (1 + b) % 2
                nxt = 1 - cur
                _wait_s(jl - 1, nxt, ss[nxt])
                _gather(jl + 1, nxt, gs[nxt])
                _wait_g(jl, cur, gs[cur])
                _scatter(jl, cur, ss[cur])
            return 0

        lax.fori_loop(0, (ln - 2) // 2, _steady, 0)
        _wait_g(ln - 1, 1, gs[1])
        _scatter(ln - 1, 1, ss[1])
        _wait_s(ln - 2, 0, ss[0])
        _wait_s(ln - 1, 1, ss[1])

    _load_win(wid * _NCH)
    pltpu.make_async_copy(hs_hbm.at[pl.ds(s * _RPT, _RPT)],
                          acc_sh.at[pl.ds(s * _RPT, _RPT)], isem).wait()
    plsc.subcore_barrier()
    _win(wid * _NCH, _WCH, loaded=True)
    _win(wid * _NCH + _WCH, _WCH)

    plsc.subcore_barrier()
    pltpu.sync_copy(acc_sh.at[pl.ds(s * _RPT, _RPT)],
                    out_hbm.at[c, pl.ds(s * _RPT, _RPT)])


def _lrelu(v):
    return jnp.where(v >= 0, v, 0.4 * v)


def _dinv_col(deg_ref, i):
    deg = (deg_ref[0] + deg_ref[1] + 1.0).reshape(1, _BLK)
    dcol = jnp.transpose(deg, (1, 0))
    row = i * _BLK + lax.broadcasted_iota(jnp.int32, (_BLK, 1), 0)
    return jnp.where(row < _N, lax.rsqrt(dcol), 0.0)


def _tc_mm_body(x_ref, win_ref, bin_ref, wc1_ref, m1_ref):
    h0 = _lrelu(jnp.dot(x_ref[...], win_ref[...],
                        preferred_element_type=jnp.float32) + bin_ref[...])
    m1_ref[...] = jnp.dot(h0, wc1_ref[...], preferred_element_type=jnp.float32)


def _tc_scale_body(m1_ref, deg_ref, hs_ref):
    i = pl.program_id(0)
    row = i * _BLK + lax.broadcasted_iota(jnp.int32, (_BLK, 1), 0)
    hs_ref[...] = jnp.where(row < _N, _dinv_col(deg_ref, i) * m1_ref[...], 0.0)


def _tc_mid_body(p_ref, hs1_ref, deg_ref, bc1_ref, wc2_ref, hs2_ref):
    dinv = _dinv_col(deg_ref, pl.program_id(0))
    acc = p_ref[0] + p_ref[1] - hs1_ref[...]
    h1 = _lrelu(dinv * acc + bc1_ref[...])
    hs2_ref[...] = dinv * jnp.dot(h1, wc2_ref[...],
                                  preferred_element_type=jnp.float32)


def _tc_out_body(p_ref, hs2_ref, deg_ref, bc2_ref, batch_ref, wout_ref,
                 bout_ref, out_ref, s_acc, cnt_acc):
    i = pl.program_id(0)

    @pl.when(i == 0)
    def _():
        s_acc[...] = jnp.zeros_like(s_acc)
        cnt_acc[...] = jnp.zeros_like(cnt_acc)

    acc = p_ref[0] + p_ref[1] - hs2_ref[...]
    h2 = _lrelu(_dinv_col(deg_ref, i) * acc + bc2_ref[...])
    n_col = i * _BLK + lax.broadcasted_iota(jnp.int32, (_G, _BLK), 1)
    mask = ((batch_ref[...] == lax.broadcasted_iota(jnp.int32, (_G, _BLK), 0))
            & (n_col < _N)).astype(jnp.float32)
    s_acc[...] += jnp.dot(mask, h2, preferred_element_type=jnp.float32)
    cnt_acc[...] += jnp.dot(mask, jnp.ones((_BLK, _D), jnp.float32),
                            preferred_element_type=jnp.float32)

    @pl.when(i == _NBLK - 1)
    def _():
        pooled = s_acc[...] / jnp.maximum(cnt_acc[...], 1.0)
        out_ref[...] = jnp.dot(pooled, wout_ref[...],
                               preferred_element_type=jnp.float32) + bout_ref[...]


_tc_mm = pl.pallas_call(
    _tc_mm_body,
    grid=(_NBLK,),
    in_specs=[
        pl.BlockSpec((_BLK, _D), lambda i: (i, 0)),
        pl.BlockSpec((_D, _D), lambda i: (0, 0)),
        pl.BlockSpec((1, _D), lambda i: (0, 0)),
        pl.BlockSpec((_D, _D), lambda i: (0, 0)),
    ],
    out_specs=pl.BlockSpec((_BLK, _D), lambda i: (i, 0)),
    out_shape=jax.ShapeDtypeStruct((_NPAD, _D), jnp.float32),
)

_tc_scale = pl.pallas_call(
    _tc_scale_body,
    grid=(_NBLK,),
    in_specs=[
        pl.BlockSpec((_BLK, _D), lambda i: (i, 0)),
        pl.BlockSpec((_NC, _BLK), lambda i: (0, i)),
    ],
    out_specs=pl.BlockSpec((_BLK, _D), lambda i: (i, 0)),
    out_shape=jax.ShapeDtypeStruct((_NPAD, _D), jnp.float32),
)

_tc_mid = pl.pallas_call(
    _tc_mid_body,
    grid=(_NBLK,),
    in_specs=[
        pl.BlockSpec((_NC, _BLK, _D), lambda i: (0, i, 0)),
        pl.BlockSpec((_BLK, _D), lambda i: (i, 0)),
        pl.BlockSpec((_NC, _BLK), lambda i: (0, i)),
        pl.BlockSpec((1, _D), lambda i: (0, 0)),
        pl.BlockSpec((_D, _D), lambda i: (0, 0)),
    ],
    out_specs=pl.BlockSpec((_BLK, _D), lambda i: (i, 0)),
    out_shape=jax.ShapeDtypeStruct((_NPAD, _D), jnp.float32),
)

_tc_out = pl.pallas_call(
    _tc_out_body,
    grid=(_NBLK,),
    in_specs=[
        pl.BlockSpec((_NC, _BLK, _D), lambda i: (0, i, 0)),
        pl.BlockSpec((_BLK, _D), lambda i: (i, 0)),
        pl.BlockSpec((_NC, _BLK), lambda i: (0, i)),
        pl.BlockSpec((1, _D), lambda i: (0, 0)),
        pl.BlockSpec((1, _BLK), lambda i: (0, i)),
        pl.BlockSpec((_D, _D), lambda i: (0, 0)),
        pl.BlockSpec((1, _D), lambda i: (0, 0)),
    ],
    out_specs=pl.BlockSpec((_G, _D), lambda i: (0, 0)),
    out_shape=jax.ShapeDtypeStruct((_G, _D), jnp.float32),
    scratch_shapes=[
        pltpu.VMEM((_G, _D), jnp.float32),
        pltpu.VMEM((_G, _D), jnp.float32),
    ],
)


def kernel(x, edge_index, batch, W_in, b_in, W_c1, b_c1, W_c2, b_c2, W_out, b_out):
    cyc = _N + jnp.arange(_NCHP * 128 - _E, dtype=jnp.int32) % (_NPAD - _N)
    ei3 = jnp.concatenate(
        [edge_index, jnp.broadcast_to(cyc, (2, cyc.shape[0]))],
        axis=1).reshape(2, _NCHP, 128)
    batch_r = batch.reshape(1, _N)

    deg = _sc_degree(ei3)
    m1 = _tc_mm(x, W_in, b_in.reshape(1, _D), W_c1)
    hs1 = _tc_scale(m1, deg)
    p1 = _sc_prop(hs1, ei3)
    hs2 = _tc_mid(p1, hs1, deg, b_c1.reshape(1, _D), W_c2)
    p2 = _sc_prop(hs2, ei3)
    return _tc_out(p2, hs2, deg, b_c2.reshape(1, _D), batch_r,
                   W_out, b_out.reshape(1, _D))


# speedup vs baseline: 36.4003x; 1.0128x over previous
import functools

import jax
import jax.numpy as jnp
from jax import lax
from jax.experimental import pallas as pl
from jax.experimental.pallas import tpu as pltpu
from jax.experimental.pallas import tpu_sc as plsc

_N = 10000
_E = 320000
_D = 128
_G = 64
_NC = 2
_NS = 16
_NW = _NC * _NS
_NPAD = 10240
_RPT = _NPAD // _NS
_NCH = 80
_WCH = 40
_NCHP = _E // 128 + 20
_BLK = 2048
_NBLK = _NPAD // _BLK

_mesh = plsc.VectorSubcoreMesh(core_axis_name="c", subcore_axis_name="s")


@functools.partial(
    pl.kernel,
    out_type=jax.ShapeDtypeStruct((_NC, _NPAD), jnp.float32),
    mesh=_mesh,
    scratch_types=[
        pltpu.VMEM((_NCH, 128), jnp.int32),
        pltpu.VMEM((_RPT,), jnp.float32),
        pltpu.VMEM((128,), jnp.float32),
        pltpu.VMEM_SHARED((_NPAD,), jnp.float32),
        pltpu.SemaphoreType.DMA,
    ],
)
def _sc_degree(ei_hbm, out_hbm, dstv, zbuf, ones, deg_sh, dsem):
    c = lax.axis_index("c")
    s = lax.axis_index("s")
    wid = c * _NS + s
    base = pl.multiple_of(wid * _NCH, 8)

    def _z(i, _):
        zbuf[pl.ds(i * 16, 16)] = jnp.zeros((16,), jnp.float32)
        return 0

    lax.fori_loop(0, _RPT // 16, _z, 0)

    def _o(i, _):
        ones[pl.ds(i * 16, 16)] = jnp.ones((16,), jnp.float32)
        return 0

    lax.fori_loop(0, 8, _o, 0)

    nch = jnp.where(wid == _NW - 1, _WCH, _NCH)
    pltpu.sync_copy(ei_hbm.at[1, pl.ds(base, _WCH)], dstv.at[pl.ds(0, _WCH)])

    @pl.when(wid != _NW - 1)
    def _():
        pltpu.sync_copy(ei_hbm.at[1, pl.ds(base + _WCH, _WCH)],
                        dstv.at[pl.ds(_WCH, _WCH)])

    pltpu.sync_copy(zbuf, deg_sh.at[pl.ds(s * _RPT, _RPT)])
    plsc.subcore_barrier()

    def _fire(j, _):
        pltpu.async_copy(ones, deg_sh.at[dstv.at[j]], dsem, add=True)
        return 0

    lax.fori_loop(0, nch, _fire, 0)

    def _drain(j, _):
        pltpu.make_async_copy(ones, deg_sh.at[dstv.at[j]], dsem).wait()
        return 0

    lax.fori_loop(0, nch, _drain, 0)
    plsc.subcore_barrier()
    pltpu.sync_copy(deg_sh.at[pl.ds(s * _RPT, _RPT)],
                    out_hbm.at[c, pl.ds(s * _RPT, _RPT)])


@functools.partial(
    pl.kernel,
    out_type=jax.ShapeDtypeStruct((_NC, _NPAD, _D), jnp.float32),
    mesh=_mesh,
    scratch_types=[
        pltpu.VMEM((_WCH, 128), jnp.int32),
        pltpu.VMEM((_WCH, 128), jnp.int32),
        pltpu.VMEM((2, 128, _D), jnp.float32),
        pltpu.VMEM_SHARED((_NPAD, _D), jnp.float32),
        pltpu.SemaphoreType.DMA,
        pltpu.SemaphoreType.DMA,
        pltpu.SemaphoreType.DMA,
        pltpu.SemaphoreType.DMA,
        pltpu.SemaphoreType.DMA,
    ],
)
def _sc_prop(hs_hbm, ei_hbm, out_hbm, srcw, dstw, buf, acc_sh,
             g0, g1, s0, s1, isem):
    c = lax.axis_index("c")
    s = lax.axis_index("s")
    wid = c * _NS + s
    gs = (g0, g1)
    ss = (s0, s1)
    pltpu.async_copy(hs_hbm.at[pl.ds(s * _RPT, _RPT)],
                     acc_sh.at[pl.ds(s * _RPT, _RPT)], isem)

    def _gather(j, b, sem):
        return pltpu.async_copy(hs_hbm.at[srcw.at[j]], buf.at[b], sem)

    def _scatter(j, b, sem):
        return pltpu.async_copy(buf.at[b], acc_sh.at[dstw.at[j]], sem,
                                add=True)

    def _wait_g(j, b, sem):
        pltpu.make_async_copy(hs_hbm.at[srcw.at[j]], buf.at[b], sem).wait()

    def _wait_s(j, b, sem):
        pltpu.make_async_copy(buf.at[b], acc_sh.at[dstw.at[j]], sem).wait()

    def _load_win(load_base):
        off = pl.multiple_of(load_base, 8)
        pltpu.sync_copy(ei_hbm.at[0, pl.ds(off, _WCH)], srcw)
        pltpu.sync_copy(ei_hbm.at[1, pl.ds(off, _WCH)], dstw)

    def _win(load_base, ln, loaded=False):
        if not loaded:
            _load_win(load_base)
        _gather(0, 0, gs[0])
        _gather(1, 1, gs[1])
        _wait_g(0, 0, gs[0])
        _scatter(0, 0, ss[0])

        def _steady(a, _):
            for b in range(2):
                jl = 1 + 2 * a + b
                cur = (1 + b) % 2
                nxt = 1 - cur
                _wait_s(jl - 1, nxt, ss[nxt])
                _gather(jl + 1, nxt, gs[nxt])
                _wait_g(jl, cur, gs[cur])
                _scatter(jl, cur, ss[cur])
            return 0

        lax.fori_loop(0, (ln - 2) // 2, _steady, 0)
        _wait_g(ln - 1, 1, gs[1])
        _scatter(ln - 1, 1, ss[1])
        _wait_s(ln - 2, 0, ss[0])
        _wait_s(ln - 1, 1, ss[1])

    _load_win(wid * _NCH)
    pltpu.make_async_copy(hs_hbm.at[pl.ds(s * _RPT, _RPT)],
                          acc_sh.at[pl.ds(s * _RPT, _RPT)], isem).wait()
    plsc.subcore_barrier()
    _win(wid * _NCH, _WCH, loaded=True)

    @pl.when(wid != _NW - 1)
    def _():
        _win(wid * _NCH + _WCH, _WCH)

    plsc.subcore_barrier()
    pltpu.sync_copy(acc_sh.at[pl.ds(s * _RPT, _RPT)],
                    out_hbm.at[c, pl.ds(s * _RPT, _RPT)])


def _lrelu(v):
    return jnp.where(v >= 0, v, 0.4 * v)


def _dinv_col(deg_ref, i):
    deg = (deg_ref[0] + deg_ref[1] + 1.0).reshape(1, _BLK)
    dcol = jnp.transpose(deg, (1, 0))
    row = i * _BLK + lax.broadcasted_iota(jnp.int32, (_BLK, 1), 0)
    return jnp.where(row < _N, lax.rsqrt(dcol), 0.0)


def _tc_mm_body(x_ref, win_ref, bin_ref, wc1_ref, m1_ref):
    h0 = _lrelu(jnp.dot(x_ref[...], win_ref[...],
                        preferred_element_type=jnp.float32) + bin_ref[...])
    m1_ref[...] = jnp.dot(h0, wc1_ref[...], preferred_element_type=jnp.float32)


def _tc_scale_body(m1_ref, deg_ref, hs_ref):
    i = pl.program_id(0)
    row = i * _BLK + lax.broadcasted_iota(jnp.int32, (_BLK, 1), 0)
    hs_ref[...] = jnp.where(row < _N, _dinv_col(deg_ref, i) * m1_ref[...], 0.0)


def _tc_mid_body(p_ref, hs1_ref, deg_ref, bc1_ref, wc2_ref, hs2_ref):
    dinv = _dinv_col(deg_ref, pl.program_id(0))
    acc = p_ref[0] + p_ref[1] - hs1_ref[...]
    h1 = _lrelu(dinv * acc + bc1_ref[...])
    hs2_ref[...] = dinv * jnp.dot(h1, wc2_ref[...],
                                  preferred_element_type=jnp.float32)


def _tc_out_body(p_ref, hs2_ref, deg_ref, bc2_ref, batch_ref, wout_ref,
                 bout_ref, out_ref, s_acc, cnt_acc):
    i = pl.program_id(0)

    @pl.when(i == 0)
    def _():
        s_acc[...] = jnp.zeros_like(s_acc)
        cnt_acc[...] = jnp.zeros_like(cnt_acc)

    acc = p_ref[0] + p_ref[1] - hs2_ref[...]
    h2 = _lrelu(_dinv_col(deg_ref, i) * acc + bc2_ref[...])
    n_col = i * _BLK + lax.broadcasted_iota(jnp.int32, (_G, _BLK), 1)
    mask = ((batch_ref[...] == lax.broadcasted_iota(jnp.int32, (_G, _BLK), 0))
            & (n_col < _N)).astype(jnp.float32)
    s_acc[...] += jnp.dot(mask, h2, preferred_element_type=jnp.float32)
    cnt_acc[...] += jnp.dot(mask, jnp.ones((_BLK, _D), jnp.float32),
                            preferred_element_type=jnp.float32)

    @pl.when(i == _NBLK - 1)
    def _():
        pooled = s_acc[...] / jnp.maximum(cnt_acc[...], 1.0)
        out_ref[...] = jnp.dot(pooled, wout_ref[...],
                               preferred_element_type=jnp.float32) + bout_ref[...]


_tc_mm = pl.pallas_call(
    _tc_mm_body,
    grid=(_NBLK,),
    in_specs=[
        pl.BlockSpec((_BLK, _D), lambda i: (i, 0)),
        pl.BlockSpec((_D, _D), lambda i: (0, 0)),
        pl.BlockSpec((1, _D), lambda i: (0, 0)),
        pl.BlockSpec((_D, _D), lambda i: (0, 0)),
    ],
    out_specs=pl.BlockSpec((_BLK, _D), lambda i: (i, 0)),
    out_shape=jax.ShapeDtypeStruct((_NPAD, _D), jnp.float32),
)

_tc_scale = pl.pallas_call(
    _tc_scale_body,
    grid=(_NBLK,),
    in_specs=[
        pl.BlockSpec((_BLK, _D), lambda i: (i, 0)),
        pl.BlockSpec((_NC, _BLK), lambda i: (0, i)),
    ],
    out_specs=pl.BlockSpec((_BLK, _D), lambda i: (i, 0)),
    out_shape=jax.ShapeDtypeStruct((_NPAD, _D), jnp.float32),
)

_tc_mid = pl.pallas_call(
    _tc_mid_body,
    grid=(_NBLK,),
    in_specs=[
        pl.BlockSpec((_NC, _BLK, _D), lambda i: (0, i, 0)),
        pl.BlockSpec((_BLK, _D), lambda i: (i, 0)),
        pl.BlockSpec((_NC, _BLK), lambda i: (0, i)),
        pl.BlockSpec((1, _D), lambda i: (0, 0)),
        pl.BlockSpec((_D, _D), lambda i: (0, 0)),
    ],
    out_specs=pl.BlockSpec((_BLK, _D), lambda i: (i, 0)),
    out_shape=jax.ShapeDtypeStruct((_NPAD, _D), jnp.float32),
)

_tc_out = pl.pallas_call(
    _tc_out_body,
    grid=(_NBLK,),
    in_specs=[
        pl.BlockSpec((_NC, _BLK, _D), lambda i: (0, i, 0)),
        pl.BlockSpec((_BLK, _D), lambda i: (i, 0)),
        pl.BlockSpec((_NC, _BLK), lambda i: (0, i)),
        pl.BlockSpec((1, _D), lambda i: (0, 0)),
        pl.BlockSpec((1, _BLK), lambda i: (0, i)),
        pl.BlockSpec((_D, _D), lambda i: (0, 0)),
        pl.BlockSpec((1, _D), lambda i: (0, 0)),
    ],
    out_specs=pl.BlockSpec((_G, _D), lambda i: (0, 0)),
    out_shape=jax.ShapeDtypeStruct((_G, _D), jnp.float32),
    scratch_shapes=[
        pltpu.VMEM((_G, _D), jnp.float32),
        pltpu.VMEM((_G, _D), jnp.float32),
    ],
)


def kernel(x, edge_index, batch, W_in, b_in, W_c1, b_c1, W_c2, b_c2, W_out, b_out):
    cyc = _N + jnp.arange(_NCHP * 128 - _E, dtype=jnp.int32) % (_NPAD - _N)
    ei3 = jnp.concatenate(
        [edge_index, jnp.broadcast_to(cyc, (2, cyc.shape[0]))],
        axis=1).reshape(2, _NCHP, 128)
    batch_r = batch.reshape(1, _N)

    deg = _sc_degree(ei3)
    m1 = _tc_mm(x, W_in, b_in.reshape(1, _D), W_c1)
    hs1 = _tc_scale(m1, deg)
    p1 = _sc_prop(hs1, ei3)
    hs2 = _tc_mid(p1, hs1, deg, b_c1.reshape(1, _D), W_c2)
    p2 = _sc_prop(hs2, ei3)
    return _tc_out(p2, hs2, deg, b_c2.reshape(1, _D), batch_r,
                   W_out, b_out.reshape(1, _D))
